# baseline (speedup 1.0000x reference)
import functools

import jax
import jax.numpy as jnp
from jax import lax
from jax.experimental import pallas as pl
from jax.experimental.pallas import tpu as pltpu
from jax.experimental.pallas import tpu_sc as plsc

DIM = 768
E = 64
K = 8
BT = 4096

NC = 2
NS = 16
NW = NC * NS


def _tc_body(x_ref, w_ref, b_ref, wout_ref, iout_ref, p_ref, *, t_total, nsteps):
    step = pl.program_id(0)

    xb = x_ref[...]
    w = w_ref[...]
    scoresT = lax.dot_general(
        w, xb, (((1,), (1,)), ((), ())), preferred_element_type=jnp.float32)

    m = jnp.max(scoresT, axis=0, keepdims=True)
    ex = jnp.exp(scoresT - m)
    probsT = ex / jnp.sum(ex, axis=0, keepdims=True)

    biasedT = probsT + b_ref[...]

    iotaF = lax.broadcasted_iota(jnp.int32, (E, BT), 0).astype(jnp.float32)
    work = biasedT
    wcols = []
    icols = []
    for _ in range(K):
        cur = jnp.max(work, axis=0, keepdims=True)
        t = jnp.where(work == cur, iotaF, float(E))
        idxF = jnp.min(t, axis=0, keepdims=True)
        onehot = (iotaF == idxF).astype(jnp.float32)
        wcols.append(jnp.sum(onehot * probsT, axis=0, keepdims=True))
        icols.append(idxF)
        work = work - onehot * 3.0e38

    wT = jnp.concatenate(wcols, axis=0)
    iT = jnp.concatenate(icols, axis=0).astype(jnp.int32)
    wout_ref[...] = wT.T
    iout_ref[...] = iT.T

    psum = jnp.sum(probsT, axis=1)

    @pl.when(step == 0)
    def _init():
        p_ref[...] = jnp.zeros_like(p_ref)

    p_ref[...] += psum[None, :]

    @pl.when(step == nsteps - 1)
    def _fin():
        p_ref[...] = p_ref[...] / t_total


def _tc_route(x, W, b2, t_total):
    nsteps = t_total // BT
    grid = (nsteps,)
    out_shapes = (
        jax.ShapeDtypeStruct((t_total, K), jnp.float32),
        jax.ShapeDtypeStruct((t_total, K), jnp.int32),
        jax.ShapeDtypeStruct((1, E), jnp.float32),
    )
    in_specs = [
        pl.BlockSpec((BT, DIM), lambda i: (i, 0)),
        pl.BlockSpec((E, DIM), lambda i: (0, 0)),
        pl.BlockSpec((E, 1), lambda i: (0, 0)),
    ]
    out_specs = (
        pl.BlockSpec((BT, K), lambda i: (i, 0)),
        pl.BlockSpec((BT, K), lambda i: (i, 0)),
        pl.BlockSpec((1, E), lambda i: (0, 0)),
    )
    return pl.pallas_call(
        functools.partial(_tc_body, t_total=t_total, nsteps=nsteps),
        grid=grid,
        in_specs=in_specs,
        out_specs=out_specs,
        out_shape=out_shapes,
    )(x, W, b2)


def _make_sc_bincount(chunk):
    mesh = plsc.VectorSubcoreMesh(core_axis_name="c", subcore_axis_name="s")

    @functools.partial(
        pl.kernel,
        out_type=jax.ShapeDtypeStruct((NW, E), jnp.float32),
        mesh=mesh,
        scratch_types=[
            pltpu.VMEM((chunk,), jnp.int32),
            pltpu.VMEM((E,), jnp.float32),
        ],
        compiler_params=pltpu.CompilerParams(needs_layout_passes=False),
    )
    def sc_bincount(idx_hbm, out_hbm, idx_v, cnt_v):
        wid = lax.axis_index("s") * NC + lax.axis_index("c")
        base = wid * chunk
        pltpu.sync_copy(idx_hbm.at[pl.ds(base, chunk)], idx_v)
        for i in range(E // 16):
            cnt_v[pl.ds(i * 16, 16)] = jnp.zeros((16,), jnp.float32)
        ones = jnp.ones((16,), jnp.float32)
        @plsc.parallel_loop(0, chunk, step=16, unroll=8)
        def body(i):
            v = idx_v[pl.ds(i, 16)]
            plsc.addupdate_scatter(cnt_v, [v], ones)
        pltpu.sync_copy(cnt_v, out_hbm.at[wid])

    return sc_bincount


def kernel(x, W, bias):
    t_total, dim = x.shape
    assert dim == DIM and W.shape == (E, DIM)
    b2 = bias.reshape(E, 1)

    weights, indices, eprobs = _tc_route(x, W, b2, t_total)

    total = t_total * K
    partial_counts = _make_sc_bincount(total // NW)(indices.reshape(-1))
    f_i = jnp.sum(partial_counts, axis=0) * (E / (K * t_total + 1e-06))

    return weights, indices, f_i, eprobs.reshape(E)


# speedup vs baseline: 1.0266x; 1.0266x over previous
import functools

import jax
import jax.numpy as jnp
from jax import lax
from jax.experimental import pallas as pl
from jax.experimental.pallas import tpu as pltpu
from jax.experimental.pallas import tpu_sc as plsc

DIM = 768
E = 64
K = 8
BT = 4096

NC = 2
NS = 16
NW = NC * NS


def _tc_body(x_ref, w_ref, b_ref, wout_ref, iout_ref, p_ref, *, t_total, nsteps):
    step = pl.program_id(0)

    xb = x_ref[...]
    w = w_ref[...]
    scoresT = lax.dot_general(
        w, xb, (((1,), (1,)), ((), ())), preferred_element_type=jnp.float32)

    m = jnp.max(scoresT, axis=0, keepdims=True)
    ex = jnp.exp(scoresT - m)
    probsT = ex / jnp.sum(ex, axis=0, keepdims=True)

    biasedT = probsT + b_ref[...]

    iotaF = lax.broadcasted_iota(jnp.int32, (E, BT), 0).astype(jnp.float32)
    work = biasedT
    wcols = []
    icols = []
    for _ in range(K):
        cur = jnp.max(work, axis=0, keepdims=True)
        t = jnp.where(work == cur, iotaF, float(E))
        idxF = jnp.min(t, axis=0, keepdims=True)
        onehot = (iotaF == idxF).astype(jnp.float32)
        wcols.append(jnp.sum(onehot * probsT, axis=0, keepdims=True))
        icols.append(idxF)
        work = work - onehot * 3.0e38

    wT = jnp.concatenate(wcols, axis=0)
    iT = jnp.concatenate(icols, axis=0).astype(jnp.int32)
    wout_ref[...] = wT.T
    iout_ref[...] = iT.T

    psum = jnp.sum(probsT, axis=1)

    @pl.when(step == 0)
    def _init():
        p_ref[...] = jnp.zeros_like(p_ref)

    p_ref[...] += psum[None, :]

    @pl.when(step == nsteps - 1)
    def _fin():
        p_ref[...] = p_ref[...] / t_total


def _tc_route(x, W, b2, t_total):
    nsteps = t_total // BT
    grid = (nsteps,)
    out_shapes = (
        jax.ShapeDtypeStruct((t_total, K), jnp.float32),
        jax.ShapeDtypeStruct((t_total, K), jnp.int32),
        jax.ShapeDtypeStruct((1, E), jnp.float32),
    )
    in_specs = [
        pl.BlockSpec((BT, DIM), lambda i: (i, 0)),
        pl.BlockSpec((E, DIM), lambda i: (0, 0)),
        pl.BlockSpec((E, 1), lambda i: (0, 0)),
    ]
    out_specs = (
        pl.BlockSpec((BT, K), lambda i: (i, 0)),
        pl.BlockSpec((BT, K), lambda i: (i, 0)),
        pl.BlockSpec((1, E), lambda i: (0, 0)),
    )
    return pl.pallas_call(
        functools.partial(_tc_body, t_total=t_total, nsteps=nsteps),
        grid=grid,
        in_specs=in_specs,
        out_specs=out_specs,
        out_shape=out_shapes,
    )(x, W, b2)


def _make_sc_bincount(chunk, scale):
    mesh = plsc.VectorSubcoreMesh(
        core_axis_name="c", subcore_axis_name="s", num_cores=1)

    @functools.partial(
        pl.kernel,
        out_type=jax.ShapeDtypeStruct((E,), jnp.float32),
        mesh=mesh,
        scratch_types=[
            pltpu.VMEM((chunk,), jnp.int32),
            pltpu.VMEM((E,), jnp.float32),
            pltpu.VMEM((NS * E,), jnp.float32),
            pltpu.VMEM_SHARED((NS * E,), jnp.float32),
        ],
        compiler_params=pltpu.CompilerParams(needs_layout_passes=False),
    )
    def sc_bincount(idx_hbm, out_hbm, idx_v, cnt_v, sums_v, shared):
        sid = lax.axis_index("s")
        base = sid * chunk
        pltpu.sync_copy(idx_hbm.at[pl.ds(base, chunk)], idx_v)
        for i in range(E // 16):
            cnt_v[pl.ds(i * 16, 16)] = jnp.zeros((16,), jnp.float32)
        ones = jnp.ones((16,), jnp.float32)

        @plsc.parallel_loop(0, chunk, step=16, unroll=8)
        def body(i):
            v = idx_v[pl.ds(i, 16)]
            plsc.addupdate_scatter(cnt_v, [v], ones)

        pltpu.sync_copy(cnt_v, shared.at[pl.ds(sid * E, E)])
        plsc.subcore_barrier()

        @pl.when(sid == 0)
        def _reduce():
            pltpu.sync_copy(shared, sums_v)
            for j in range(E // 16):
                acc = jnp.zeros((16,), jnp.float32)
                for r in range(NS):
                    acc = acc + sums_v[pl.ds(r * E + j * 16, 16)]
                cnt_v[pl.ds(j * 16, 16)] = acc * scale
            pltpu.sync_copy(cnt_v, out_hbm)

    return sc_bincount


def kernel(x, W, bias):
    t_total, dim = x.shape
    assert dim == DIM and W.shape == (E, DIM)
    b2 = bias.reshape(E, 1)

    weights, indices, eprobs = _tc_route(x, W, b2, t_total)

    total = t_total * K
    scale = E / (K * t_total + 1e-06)
    f_i = _make_sc_bincount(total // NS, scale)(indices.reshape(-1))

    return weights, indices, f_i, eprobs.reshape(E)
